# baseline (speedup 1.0000x reference)
import functools

import jax
import jax.numpy as jnp
from jax import lax
from jax.experimental import pallas as pl
from jax.experimental.pallas import tpu as pltpu
from jax.experimental.pallas import tpu_sc as plsc

NC = 2
NS = 16
LANES = 16
NW = NC * NS
IDXB = 128
TR = 8
TCOL = 128
TPAD = TCOL + 1


def _sc_transpose(y4, *, ntile, D):
    nrt = D // TR
    per_w = (ntile + NW - 1) // NW
    half = (per_w + 1) // 2
    mesh = plsc.VectorSubcoreMesh(
        core_axis_name="c", subcore_axis_name="s", num_cores=NC, num_subcores=NS
    )

    @functools.partial(
        pl.kernel,
        out_type=jax.ShapeDtypeStruct((ntile * TCOL, D), jnp.float32),
        mesh=mesh,
        scratch_types=[
            pltpu.VMEM((2, nrt, TR, TCOL), jnp.float32),
            pltpu.VMEM((2, TCOL, D + 1), jnp.float32),
            pltpu.SemaphoreType.DMA,
            pltpu.SemaphoreType.DMA,
            pltpu.SemaphoreType.DMA,
            pltpu.SemaphoreType.DMA,
        ],
        compiler_params=pltpu.CompilerParams(
            use_tc_tiling_on_sc=False, needs_layout_passes=False
        ),
    )
    def ka(y_hbm, lin_hbm, ybuf, lbuf, sy0, sy1, sl0, sl1):
        wid = lax.axis_index("s") * NC + lax.axis_index("c")
        t_base = wid * per_w
        t_last = ntile - 1
        iota = lax.iota(jnp.int32, LANES)
        jvecs = [iota + g * LANES for g in range(TCOL // LANES)]

        def tile_of(k):
            return jnp.minimum(t_base + k, t_last)

        def fire_reads(k, buf, sem):
            t = tile_of(k)
            for r in range(nrt):
                pltpu.async_copy(y_hbm.at[r, t], ybuf.at[buf, r], sem)

        def drain_reads(buf, sem):
            pltpu.make_async_copy(
                y_hbm.at[pl.ds(0, nrt), 0], ybuf.at[buf], sem
            ).wait()

        def fire_out(k, buf, sem):
            t = tile_of(k)
            pltpu.async_copy(
                lbuf.at[buf, :, pl.ds(0, D)],
                lin_hbm.at[pl.ds(t * TCOL, TCOL)],
                sem,
            )

        def drain_out(buf, sem):
            pltpu.make_async_copy(
                lin_hbm.at[pl.ds(0, TCOL)], lbuf.at[buf, :, pl.ds(0, D)], sem
            ).wait()

        def transpose(buf):
            for r in range(nrt):
                for i in range(TR):
                    d = TR * r + i
                    dvec = lax.broadcast(jnp.int32(d), (LANES,))
                    for g in range(TCOL // LANES):
                        x = ybuf[buf, r, i, pl.ds(g * LANES, LANES)]
                        plsc.store_scatter(lbuf.at[buf], [jvecs[g], dvec], x)

        fire_reads(0, 0, sy0)

        def m_body(m, carry):
            k0 = 2 * m
            fire_reads(k0 + 1, 1, sy1)
            drain_reads(0, sy0)

            @pl.when(m > 0)
            def _():
                drain_out(0, sl0)

            transpose(0)
            fire_out(k0, 0, sl0)
            fire_reads(k0 + 2, 0, sy0)
            drain_reads(1, sy1)

            @pl.when(m > 0)
            def _():
                drain_out(1, sl1)

            transpose(1)
            fire_out(k0 + 1, 1, sl1)
            return carry

        lax.fori_loop(0, half, m_body, 0)
        drain_reads(0, sy0)
        drain_out(0, sl0)
        drain_out(1, sl1)

    return ka(y4)


def _sc_embed(table, tok_t, vw, *, V, D, B, L):
    bw = B // NW
    nb = bw // IDXB
    nr = D // TR
    nct = bw // TCOL
    scale = 1.0 / (V - 1)
    mesh = plsc.VectorSubcoreMesh(
        core_axis_name="c", subcore_axis_name="s", num_cores=NC, num_subcores=NS
    )

    @functools.partial(
        pl.kernel,
        out_type=jax.ShapeDtypeStruct((L, nr, B // TCOL, TR, TCOL), jnp.float32),
        mesh=mesh,
        scratch_types=[
            pltpu.VMEM((L, bw), jnp.int32),
            pltpu.VMEM((2, bw, D), jnp.float32),
            pltpu.VMEM((2, nr, nct, TR, TPAD), jnp.float32),
            pltpu.VMEM((D,), jnp.float32),
            pltpu.SemaphoreType.DMA,
            pltpu.SemaphoreType.DMA,
            pltpu.SemaphoreType.DMA,
            pltpu.SemaphoreType.DMA,
        ],
        compiler_params=pltpu.CompilerParams(
            use_tc_tiling_on_sc=False, needs_layout_passes=False
        ),
    )
    def k(table_hbm, tok_hbm, vw_hbm, out_hbm, tokb_v, rows_v, tr_v, vw_v,
          sg0, sg1, so0, so1):
        wid = lax.axis_index("s") * NC + lax.axis_index("c")
        b0 = wid * bw
        ct0 = wid * nct
        pltpu.sync_copy(vw_hbm, vw_v)
        pltpu.sync_copy(tok_hbm.at[:, pl.ds(b0, bw)], tokb_v)
        vwlo = vw_v[pl.ds(0, LANES)]
        vwhi = vw_v[pl.ds(LANES, LANES)]
        iota = lax.iota(jnp.int32, LANES)
        rlo = lax.shift_right_logical(iota, 3)
        rhi = rlo + (LANES // TR)
        ivec = lax.bitwise_and(iota, TR - 1)

        def fire_gathers(l, buf, sem):
            for j in range(nb):
                pltpu.async_copy(
                    table_hbm.at[tokb_v.at[l, pl.ds(j * IDXB, IDXB)]],
                    rows_v.at[buf, pl.ds(j * IDXB, IDXB)],
                    sem,
                )

        def drain_gathers(buf, sem):
            pltpu.make_async_copy(
                table_hbm.at[pl.ds(0, bw)], rows_v.at[buf], sem
            ).wait()

        def fire_out(l, buf, sem):
            for r in range(nr):
                pltpu.async_copy(
                    tr_v.at[buf, r, :, :, pl.ds(0, TCOL)],
                    out_hbm.at[l, r, pl.ds(ct0, nct)],
                    sem,
                )

        def drain_out(buf, sem):
            for r in range(nr):
                pltpu.make_async_copy(
                    out_hbm.at[0, 0, pl.ds(0, nct)],
                    tr_v.at[buf, r, :, :, pl.ds(0, TCOL)],
                    sem,
                ).wait()

        def compute(l, buf):
            def tok_body(t, c2):
                tok16 = tokb_v[l, pl.ds(t * LANES, LANES)]
                vals = tok16.astype(jnp.float32) * scale
                cvec = lax.broadcast(lax.shift_right_logical(t * LANES, 7), (LANES,))
                for q in range(LANES):
                    b = t * LANES + q
                    jvec = lax.broadcast(lax.bitwise_and(b, TCOL - 1), (LANES,))
                    lo = rows_v[buf, b, pl.ds(0, LANES)] + vals[q] * vwlo
                    hi = rows_v[buf, b, pl.ds(LANES, LANES)] + vals[q] * vwhi
                    plsc.store_scatter(tr_v.at[buf], [rlo, cvec, ivec, jvec], lo)
                    plsc.store_scatter(tr_v.at[buf], [rhi, cvec, ivec, jvec], hi)
                return c2

            lax.fori_loop(0, bw // LANES, tok_body, 0)

        fire_gathers(0, 0, sg0)

        def g_body(g, carry):
            l0 = 2 * g
            l1 = 2 * g + 1
            fire_gathers(l1, 1, sg1)
            drain_gathers(0, sg0)

            @pl.when(g > 0)
            def _():
                drain_out(0, so0)

            compute(l0, 0)
            fire_out(l0, 0, so0)
            fire_gathers(jnp.minimum(l1 + 1, L - 1), 0, sg0)
            drain_gathers(1, sg1)

            @pl.when(g > 0)
            def _():
                drain_out(1, so1)

            compute(l1, 1)
            fire_out(l1, 1, so1)
            return carry

        lax.fori_loop(0, L // 2, g_body, 0)
        drain_gathers(0, sg0)
        drain_out(0, so0)
        drain_out(1, so1)

    return k(table, tok_t, vw)


def kernel(tokens, index_weight, value_w, token_values):
    B, L = tokens.shape
    V, D = index_weight.shape
    tok_t = tokens.T.astype(jnp.int32)
    vw = value_w.reshape(D)
    ntile = (V + TCOL - 1) // TCOL
    vpad = ntile * TCOL
    y4 = (
        jnp.pad(index_weight, ((0, vpad - V), (0, 0)))
        .reshape(ntile, TCOL, D // TR, TR)
        .transpose(2, 0, 3, 1)
    )
    tablin = _sc_transpose(y4, ntile=ntile, D=D)
    out5 = _sc_embed(tablin, tok_t, vw, V=V, D=D, B=B, L=L)
    out_t = out5.transpose(0, 1, 3, 2, 4).reshape(L, D, B)
    return out_t.transpose(2, 0, 1)


# speedup vs baseline: 1.2298x; 1.2298x over previous
import functools

import jax
import jax.numpy as jnp
from jax import lax
from jax.experimental import pallas as pl
from jax.experimental.pallas import tpu as pltpu
from jax.experimental.pallas import tpu_sc as plsc

NC = 2
NS = 16
LANES = 16
NW = NC * NS
IDXB = 128
TR = 8
TCOL = 128
TPAD = TCOL + 1


def _sc_transpose(y4, *, ntile, D):
    nrt = D // TR
    G = 4
    per_w = (ntile + NW - 1) // NW
    half = (per_w + 2 * G - 1) // (2 * G)
    mesh = plsc.VectorSubcoreMesh(
        core_axis_name="c", subcore_axis_name="s", num_cores=NC, num_subcores=NS
    )

    @functools.partial(
        pl.kernel,
        out_type=jax.ShapeDtypeStruct((ntile * TCOL, D), jnp.float32),
        mesh=mesh,
        scratch_types=[
            pltpu.VMEM((2, nrt, G + 1, TR, TCOL + 1), jnp.float32),
            pltpu.VMEM((2, G * TCOL, D), jnp.float32),
            pltpu.SemaphoreType.DMA,
            pltpu.SemaphoreType.DMA,
            pltpu.SemaphoreType.DMA,
            pltpu.SemaphoreType.DMA,
        ],
        compiler_params=pltpu.CompilerParams(
            use_tc_tiling_on_sc=False, needs_layout_passes=False
        ),
    )
    def ka(y_hbm, lin_hbm, ybuf, lbuf, sy0, sy1, sl0, sl1):
        wid = lax.axis_index("s") * NC + lax.axis_index("c")
        t_base = wid * per_w
        t_last = ntile - G
        iota = lax.iota(jnp.int32, LANES)
        ivec = lax.bitwise_and(iota, TR - 1)
        rvec_lo = lax.shift_right_logical(iota, 3)
        rvec_hi = rvec_lo + (LANES // TR)

        def tile_of(k):
            return jnp.minimum(t_base + k * G, t_last)

        def fire_reads(k, buf, sem):
            t = tile_of(k)
            for r in range(nrt):
                pltpu.async_copy(
                    y_hbm.at[r, pl.ds(t, G)],
                    ybuf.at[buf, r, pl.ds(0, G), :, pl.ds(0, TCOL)],
                    sem,
                )

        def drain_reads(buf, sem):
            for r in range(nrt):
                pltpu.make_async_copy(
                    y_hbm.at[r, pl.ds(0, G)],
                    ybuf.at[buf, r, pl.ds(0, G), :, pl.ds(0, TCOL)],
                    sem,
                ).wait()

        def fire_out(k, buf, sem):
            t = tile_of(k)
            pltpu.async_copy(lbuf.at[buf], lin_hbm.at[pl.ds(t * TCOL, G * TCOL)], sem)

        def drain_out(buf, sem):
            pltpu.make_async_copy(
                lin_hbm.at[pl.ds(0, G * TCOL)], lbuf.at[buf], sem
            ).wait()

        def transpose(buf):
            yb = ybuf.at[buf]

            def t_body(tl, c2):
                tvec = lax.broadcast(tl, (LANES,))
                v0 = tl * TCOL

                def j_body(j, c3):
                    jvec = lax.broadcast(j, (LANES,))
                    lo = plsc.load_gather(yb, [rvec_lo, tvec, ivec, jvec])
                    hi = plsc.load_gather(yb, [rvec_hi, tvec, ivec, jvec])
                    lbuf[buf, v0 + j, pl.ds(0, LANES)] = lo
                    lbuf[buf, v0 + j, pl.ds(LANES, LANES)] = hi
                    return c3

                lax.fori_loop(0, TCOL, j_body, 0, unroll=16)
                return c2

            lax.fori_loop(0, G, t_body, 0)

        fire_reads(0, 0, sy0)

        def m_body(m, carry):
            k0 = 2 * m
            fire_reads(k0 + 1, 1, sy1)
            drain_reads(0, sy0)

            @pl.when(m > 0)
            def _():
                drain_out(0, sl0)

            transpose(0)
            fire_out(k0, 0, sl0)
            fire_reads(k0 + 2, 0, sy0)
            drain_reads(1, sy1)

            @pl.when(m > 0)
            def _():
                drain_out(1, sl1)

            transpose(1)
            fire_out(k0 + 1, 1, sl1)
            return carry

        lax.fori_loop(0, half, m_body, 0)
        drain_reads(0, sy0)
        drain_out(0, sl0)
        drain_out(1, sl1)

    return ka(y4)


def _sc_embed(table, tok_t, vw, *, V, D, B, L):
    bw = B // NW
    nb = bw // IDXB
    nr = D // TR
    nct = bw // TCOL
    scale = 1.0 / (V - 1)
    mesh = plsc.VectorSubcoreMesh(
        core_axis_name="c", subcore_axis_name="s", num_cores=NC, num_subcores=NS
    )

    @functools.partial(
        pl.kernel,
        out_type=jax.ShapeDtypeStruct((L, nr, B // TCOL, TR, TCOL), jnp.float32),
        mesh=mesh,
        scratch_types=[
            pltpu.VMEM((L, bw), jnp.int32),
            pltpu.VMEM((2, bw, D), jnp.float32),
            pltpu.VMEM((2, nr, nct, TR, TPAD), jnp.float32),
            pltpu.VMEM((D,), jnp.float32),
            pltpu.SemaphoreType.DMA,
            pltpu.SemaphoreType.DMA,
            pltpu.SemaphoreType.DMA,
            pltpu.SemaphoreType.DMA,
        ],
        compiler_params=pltpu.CompilerParams(
            use_tc_tiling_on_sc=False, needs_layout_passes=False
        ),
    )
    def k(table_hbm, tok_hbm, vw_hbm, out_hbm, tokb_v, rows_v, tr_v, vw_v,
          sg0, sg1, so0, so1):
        wid = lax.axis_index("s") * NC + lax.axis_index("c")
        b0 = wid * bw
        ct0 = wid * nct
        pltpu.sync_copy(vw_hbm, vw_v)
        pltpu.sync_copy(tok_hbm.at[:, pl.ds(b0, bw)], tokb_v)
        vwlo = vw_v[pl.ds(0, LANES)]
        vwhi = vw_v[pl.ds(LANES, LANES)]
        iota = lax.iota(jnp.int32, LANES)
        rlo = lax.shift_right_logical(iota, 3)
        rhi = rlo + (LANES // TR)
        ivec = lax.bitwise_and(iota, TR - 1)

        def fire_gathers(l, buf, sem):
            for j in range(nb):
                pltpu.async_copy(
                    table_hbm.at[tokb_v.at[l, pl.ds(j * IDXB, IDXB)]],
                    rows_v.at[buf, pl.ds(j * IDXB, IDXB)],
                    sem,
                )

        def drain_gathers(buf, sem):
            pltpu.make_async_copy(
                table_hbm.at[pl.ds(0, bw)], rows_v.at[buf], sem
            ).wait()

        def fire_out(l, buf, sem):
            for r in range(nr):
                pltpu.async_copy(
                    tr_v.at[buf, r, :, :, pl.ds(0, TCOL)],
                    out_hbm.at[l, r, pl.ds(ct0, nct)],
                    sem,
                )

        def drain_out(buf, sem):
            for r in range(nr):
                pltpu.make_async_copy(
                    out_hbm.at[0, 0, pl.ds(0, nct)],
                    tr_v.at[buf, r, :, :, pl.ds(0, TCOL)],
                    sem,
                ).wait()

        def compute(l, buf):
            def tok_body(t, c2):
                tok16 = tokb_v[l, pl.ds(t * LANES, LANES)]
                vals = tok16.astype(jnp.float32) * scale
                cvec = lax.broadcast(lax.shift_right_logical(t * LANES, 7), (LANES,))
                for q in range(LANES):
                    b = t * LANES + q
                    jvec = lax.broadcast(lax.bitwise_and(b, TCOL - 1), (LANES,))
                    lo = rows_v[buf, b, pl.ds(0, LANES)] + vals[q] * vwlo
                    hi = rows_v[buf, b, pl.ds(LANES, LANES)] + vals[q] * vwhi
                    plsc.store_scatter(tr_v.at[buf], [rlo, cvec, ivec, jvec], lo)
                    plsc.store_scatter(tr_v.at[buf], [rhi, cvec, ivec, jvec], hi)
                return c2

            lax.fori_loop(0, bw // LANES, tok_body, 0)

        fire_gathers(0, 0, sg0)

        def g_body(g, carry):
            l0 = 2 * g
            l1 = 2 * g + 1
            fire_gathers(l1, 1, sg1)
            drain_gathers(0, sg0)

            @pl.when(g > 0)
            def _():
                drain_out(0, so0)

            compute(l0, 0)
            fire_out(l0, 0, so0)
            fire_gathers(jnp.minimum(l1 + 1, L - 1), 0, sg0)
            drain_gathers(1, sg1)

            @pl.when(g > 0)
            def _():
                drain_out(1, so1)

            compute(l1, 1)
            fire_out(l1, 1, so1)
            return carry

        lax.fori_loop(0, L // 2, g_body, 0)
        drain_gathers(0, sg0)
        drain_out(0, so0)
        drain_out(1, so1)

    return k(table, tok_t, vw)


def kernel(tokens, index_weight, value_w, token_values):
    B, L = tokens.shape
    V, D = index_weight.shape
    tok_t = tokens.T.astype(jnp.int32)
    vw = value_w.reshape(D)
    ntile = (V + TCOL - 1) // TCOL
    vpad = ntile * TCOL
    y4 = (
        jnp.pad(index_weight, ((0, vpad - V), (0, 0)))
        .reshape(ntile, TCOL, D // TR, TR)
        .transpose(2, 0, 3, 1)
    )
    tablin = _sc_transpose(y4, ntile=ntile, D=D)
    out5 = _sc_embed(tablin, tok_t, vw, V=V, D=D, B=B, L=L)
    out_t = out5.transpose(0, 1, 3, 2, 4).reshape(L, D, B)
    return out_t.transpose(2, 0, 1)
